# baseline (speedup 1.0000x reference)
import functools

import jax
import jax.numpy as jnp
from jax import lax
from jax.experimental import pallas as pl
from jax.experimental.pallas import tpu as pltpu
from jax.experimental.pallas import tpu_sc as plsc

_NC, _NS, _L = 2, 16, 16
_NW = _NC * _NS
_NSL = 8
_SW = 16


def _make_tri_gather(T, E, D, C):
    per_w = T // _NW
    n_chunks = per_w // C
    assert per_w * _NW == T and n_chunks * C == per_w and C % 8 == 0

    mesh = plsc.VectorSubcoreMesh(core_axis_name="c", subcore_axis_name="s")

    @functools.partial(
        pl.kernel,
        mesh=mesh,
        compiler_params=pltpu.CompilerParams(use_tc_tiling_on_sc=False),
        out_type=jax.ShapeDtypeStruct((_NSL, T, _SW), jnp.float32),
        scratch_types=[
            pltpu.VMEM((C,), jnp.int32),
            pltpu.VMEM((C,), jnp.int32),
            pltpu.VMEM((C,), jnp.int32),
            pltpu.VMEM((C, D), jnp.float32),
            pltpu.VMEM((C, D), jnp.float32),
            pltpu.VMEM((C, D), jnp.float32),
            pltpu.VMEM((_NSL, C, _SW), jnp.float32),
            pltpu.SemaphoreType.DMA,
        ],
    )
    def tri_gather(x_hbm, e0_hbm, e1_hbm, e2_hbm, t_hbm,
                   i0_v, i1_v, i2_v, g0_v, g1_v, g2_v, ts_v, sem):
        wid = lax.axis_index("s") * _NC + lax.axis_index("c")
        base = wid * per_w

        def chunk(j, carry):
            off = base + j * C
            pltpu.sync_copy(e0_hbm.at[pl.ds(off, C)], i0_v)
            pltpu.sync_copy(e1_hbm.at[pl.ds(off, C)], i1_v)
            pltpu.sync_copy(e2_hbm.at[pl.ds(off, C)], i2_v)
            cp0 = pltpu.async_copy(x_hbm.at[i0_v], g0_v, sem)
            cp1 = pltpu.async_copy(x_hbm.at[i1_v], g1_v, sem)
            cp2 = pltpu.async_copy(x_hbm.at[i2_v], g2_v, sem)
            cp0.wait()
            cp1.wait()
            cp2.wait()

            def row(r, c2):
                for l in range(D // _L):
                    sl = pl.ds(l * _L, _L)
                    ts_v[l, r, pl.ds(0, _SW)] = g0_v[r, sl] - g1_v[r, sl] + g2_v[r, sl]
                return c2

            lax.fori_loop(0, C, row, 0, unroll=False)
            for s in range(_NSL):
                pltpu.sync_copy(ts_v.at[s], t_hbm.at[s, pl.ds(off, C)])
            return carry

        lax.fori_loop(0, n_chunks, chunk, 0, unroll=False)

    return tri_gather


def _make_node_scatter(E, N, D, C):
    per_w = E // _NW
    n_chunks = per_w // C
    per_t = N // _NS
    assert per_w * _NW == E and n_chunks * C == per_w and C % 8 == 0
    assert per_t * _NS == N and per_t % 8 == 0
    DC = 128
    assert per_t % DC == 0
    n_dump = per_t // DC

    mesh = plsc.VectorSubcoreMesh(core_axis_name="c", subcore_axis_name="s")

    @functools.partial(
        pl.kernel,
        mesh=mesh,
        out_type=jax.ShapeDtypeStruct((_NC, N, D), jnp.float32),
        scratch_types=[
            pltpu.VMEM_SHARED((N, D), jnp.float32),
            pltpu.VMEM((C,), jnp.int32),
            pltpu.VMEM((C,), jnp.int32),
            pltpu.VMEM((C, D), jnp.float32),
            pltpu.VMEM((DC, D), jnp.float32),
            pltpu.SemaphoreType.DMA,
        ],
    )
    def node_scatter(x_hbm, src_hbm, dst_hbm, u_hbm, acc_sh, is_v, id_v, bx_v, db_v, sem):
        cid = lax.axis_index("c")
        tid = lax.axis_index("s")
        wid = tid * _NC + cid

        def zrow(r, c2):
            for l in range(D // _L):
                db_v[r, pl.ds(l * _L, _L)] = jnp.zeros((_L,), jnp.float32)
            return c2

        lax.fori_loop(0, DC, zrow, 0, unroll=False)

        def zchunk(q, c2):
            pltpu.sync_copy(db_v, acc_sh.at[pl.ds(tid * per_t + q * DC, DC)])
            return c2

        lax.fori_loop(0, n_dump, zchunk, 0, unroll=False)
        plsc.subcore_barrier()

        base = wid * per_w

        def chunk(j, carry):
            off = base + j * C
            pltpu.sync_copy(src_hbm.at[pl.ds(off, C)], is_v)
            pltpu.sync_copy(dst_hbm.at[pl.ds(off, C)], id_v)
            pltpu.async_copy(x_hbm.at[pl.ds(off, C)], bx_v, sem).wait()
            pltpu.sync_copy(bx_v, acc_sh.at[id_v], add=True)

            def nrow(r, c2):
                for l in range(D // _L):
                    sl = pl.ds(l * _L, _L)
                    bx_v[r, sl] = -bx_v[r, sl]
                return c2

            lax.fori_loop(0, C, nrow, 0, unroll=False)
            pltpu.sync_copy(bx_v, acc_sh.at[is_v], add=True)
            return carry

        lax.fori_loop(0, n_chunks, chunk, 0, unroll=False)
        plsc.subcore_barrier()

        def dump(q, c2):
            r0 = tid * per_t + q * DC
            pltpu.sync_copy(acc_sh.at[pl.ds(r0, DC)], db_v)
            pltpu.sync_copy(db_v, u_hbm.at[cid, pl.ds(r0, DC)])
            return c2

        lax.fori_loop(0, n_dump, dump, 0, unroll=False)

    return node_scatter


def _make_edge_gather(E, N, D, C):
    per_w = E // _NW
    n_chunks = per_w // C
    assert per_w * _NW == E and n_chunks * C == per_w and C % 8 == 0

    mesh = plsc.VectorSubcoreMesh(core_axis_name="c", subcore_axis_name="s")

    @functools.partial(
        pl.kernel,
        mesh=mesh,
        out_type=jax.ShapeDtypeStruct((E, D), jnp.float32),
        scratch_types=[
            pltpu.VMEM((C,), jnp.int32),
            pltpu.VMEM((C,), jnp.int32),
            pltpu.VMEM((C, D), jnp.float32),
            pltpu.VMEM((C, D), jnp.float32),
            pltpu.SemaphoreType.DMA,
        ],
    )
    def edge_gather(up_hbm, src_hbm, dst_hbm, z_hbm, is_v, id_v, gs_v, gd_v, sem):
        wid = lax.axis_index("s") * _NC + lax.axis_index("c")
        base = wid * per_w

        def chunk(j, carry):
            off = base + j * C
            pltpu.sync_copy(src_hbm.at[pl.ds(off, C)], is_v)
            pltpu.sync_copy(dst_hbm.at[pl.ds(off, C)], id_v)
            cps = pltpu.async_copy(up_hbm.at[is_v], gs_v, sem)
            cpd = pltpu.async_copy(up_hbm.at[id_v], gd_v, sem)
            cps.wait()
            cpd.wait()

            def row(r, c2):
                for l in range(D // _L):
                    sl = pl.ds(l * _L, _L)
                    gd_v[r, sl] = gd_v[r, sl] - gs_v[r, sl]
                return c2

            lax.fori_loop(0, C, row, 0, unroll=False)
            pltpu.sync_copy(gd_v, z_hbm.at[pl.ds(off, C)])
            return carry

        lax.fori_loop(0, n_chunks, chunk, 0, unroll=False)

    return edge_gather


def _make_tri_scatter(T, E_pad, D, P, C):
    n_pass = E_pad // P
    n_unit = n_pass * _NSL
    per_t = T // _NS
    n_chunks = per_t // C
    nv = C // _L
    pt_rows = P // _NS
    DC = 128
    assert n_pass * P == E_pad and n_unit % _NC == 0
    assert per_t * _NS == T and n_chunks * C == per_t and C % _L == 0
    assert nv * _L == C
    assert pt_rows * _NS == P and pt_rows % DC == 0
    n_dump = pt_rows // DC
    SACR = P

    mesh = plsc.VectorSubcoreMesh(core_axis_name="c", subcore_axis_name="s")

    @functools.partial(
        pl.kernel,
        mesh=mesh,
        compiler_params=pltpu.CompilerParams(use_tc_tiling_on_sc=False),
        out_type=jax.ShapeDtypeStruct((_NSL, E_pad, _SW), jnp.float32),
        scratch_types=[
            pltpu.VMEM_SHARED((P + 8, _SW), jnp.float32),
            pltpu.VMEM((C,), jnp.int32),
            pltpu.VMEM((C,), jnp.int32),
            pltpu.VMEM((C,), jnp.int32),
            pltpu.VMEM((C,), jnp.int32),
            pltpu.VMEM((C, _SW), jnp.float32),
            pltpu.VMEM((DC, _SW), jnp.float32),
            pltpu.VMEM((DC, _SW), jnp.float32),
            pltpu.SemaphoreType.DMA,
        ],
    )
    def tri_scatter(t_hbm, e0_hbm, e1_hbm, e2_hbm, z_hbm,
                    acc_sh, ie_v, c0_v, c1_v, c2_v, bt_v, zb_v, db_v, sem):
        cid = lax.axis_index("c")
        tid = lax.axis_index("s")
        tri0 = tid * per_t

        def zrow(r, c2):
            zb_v[r, pl.ds(0, _L)] = jnp.zeros((_L,), jnp.float32)
            return c2

        lax.fori_loop(0, DC, zrow, 0, unroll=False)

        def run_unit(uu, carry):
            u = uu * _NC + cid
            p = u // _NSL
            s = u - p * _NSL
            pass_base = p * P

            def zchunk(q, c2):
                pltpu.sync_copy(zb_v, acc_sh.at[pl.ds(tid * pt_rows + q * DC, DC)])
                return c2

            lax.fori_loop(0, n_dump, zchunk, 0, unroll=False)
            plsc.subcore_barrier()

            def chunk(j, carry2):
                off = tri0 + j * C

                def clamp_into(e_hbm, cl_v):
                    pltpu.sync_copy(e_hbm.at[pl.ds(off, C)], ie_v)

                    def grp(g, c3):
                        sl = pl.ds(g * _L, _L)
                        e = ie_v[sl]
                        rel = e - jnp.full((_L,), pass_base, jnp.int32)
                        m = (rel >= jnp.full((_L,), 0, jnp.int32)) & (
                            rel < jnp.full((_L,), P, jnp.int32))
                        cl_v[sl] = jnp.where(m, rel, jnp.full((_L,), SACR, jnp.int32))
                        return c3

                    lax.fori_loop(0, nv, grp, 0, unroll=False)

                clamp_into(e0_hbm, c0_v)
                clamp_into(e1_hbm, c1_v)
                clamp_into(e2_hbm, c2_v)
                pltpu.async_copy(t_hbm.at[s, pl.ds(off, C)], bt_v, sem).wait()
                pltpu.sync_copy(bt_v, acc_sh.at[c0_v], add=True)
                pltpu.sync_copy(bt_v, acc_sh.at[c2_v], add=True)

                def nrow(r, c3):
                    bt_v[r, pl.ds(0, _L)] = -bt_v[r, pl.ds(0, _L)]
                    return c3

                lax.fori_loop(0, C, nrow, 0, unroll=False)
                pltpu.sync_copy(bt_v, acc_sh.at[c1_v], add=True)
                return carry2

            lax.fori_loop(0, n_chunks, chunk, 0, unroll=False)
            plsc.subcore_barrier()

            def dump(q, c2):
                r0 = tid * pt_rows + q * DC
                pltpu.sync_copy(acc_sh.at[pl.ds(r0, DC)], db_v)
                pltpu.sync_copy(db_v, z_hbm.at[s, pl.ds(pass_base + r0, DC)])
                return c2

            lax.fori_loop(0, n_dump, dump, 0, unroll=False)
            plsc.subcore_barrier()
            return carry

        lax.fori_loop(0, n_unit // _NC, run_unit, 0, unroll=False)

    return tri_scatter


def _combine_body(z1_ref, x_ref, z3_ref, w2_ref, w1_ref, o_ref):
    acc = jnp.dot(x_ref[...], w1_ref[...], preferred_element_type=jnp.float32)
    for s in range(_NSL):
        acc += jnp.dot(z1_ref[s], w2_ref[s * _SW:(s + 1) * _SW, :],
                       preferred_element_type=jnp.float32)
    o_ref[...] = jnp.tanh(acc + z3_ref[...])


def kernel(x, edge_index, tri_index, weight_0, weight_1, weight_2):
    n_edges, d = x.shape
    n_tri = tri_index.shape[1]
    src, dst = edge_index[0], edge_index[1]
    e0, e1, e2 = tri_index[0], tri_index[1], tri_index[2]

    u2 = _make_node_scatter(n_edges, 10240, d, 200)(x, src, dst)
    up = (u2[0, :10000] + u2[1, :10000]) @ weight_0
    z3 = _make_edge_gather(n_edges, 10000, d, 200)(up, src, dst)

    t = _make_tri_gather(n_tri, n_edges, d, 200)(x, e0, e1, e2)
    z1 = _make_tri_scatter(n_tri, 327680, d, 81920, 2000)(t, e0, e1, e2)

    blk = 640
    return pl.pallas_call(
        _combine_body,
        grid=(n_edges // blk,),
        in_specs=[
            pl.BlockSpec((_NSL, blk, _SW), lambda i: (0, i, 0)),
            pl.BlockSpec((blk, d), lambda i: (i, 0)),
            pl.BlockSpec((blk, d), lambda i: (i, 0)),
            pl.BlockSpec((d, d), lambda i: (0, 0)),
            pl.BlockSpec((d, d), lambda i: (0, 0)),
        ],
        out_specs=pl.BlockSpec((blk, d), lambda i: (i, 0)),
        out_shape=jax.ShapeDtypeStruct((n_edges, d), x.dtype),
    )(z1, x, z3, weight_2, weight_1)
